# baseline (speedup 1.0000x reference)
import functools

import jax
import jax.numpy as jnp
from jax.experimental import pallas as pl
from jax.experimental.pallas import tpu as pltpu

_BS = 128
_BT = 512


def _nsoftplus(p):
    return -(jnp.maximum(p, 0.0) + jnp.log1p(jnp.exp(-jnp.abs(p))))


def _nt(a, b):
    return jax.lax.dot_general(a, b, (((1,), (1,)), ((), ())),
                               preferred_element_type=jnp.float32)


def _mm(a, b):
    return jax.lax.dot_general(a, b, (((1,), (0,)), ((), ())),
                               preferred_element_type=jnp.float32)


def _scan_kernel(topi_ref, x_ref,
                 qU0, qU1, kU0, kU1, vU0, vU1, gU0, gU1,
                 qV0, qV1, kV0, kV1, vV0, vV1, gV0, gV1,
                 ww_ref, dW_ref, db_ref, dbT_ref, rms_ref,
                 o_ref, st_ref, *, n_heads, d_head):
    del topi_ref
    sblk = pl.program_id(0)
    bsz = x_ref.shape[0]
    bs = x_ref.shape[1]
    m = bsz * bs
    hid = n_heads * d_head

    x = jnp.concatenate([x_ref[b] for b in range(bsz)], axis=0)

    rows = jax.lax.broadcasted_iota(jnp.int32, (m, m), 0)
    cols = jax.lax.broadcasted_iota(jnp.int32, (m, m), 1)
    bd = (rows >= cols) & (rows - cols < bs) & ((rows >= bs) == (cols >= bs))
    bdf = bd.astype(jnp.float32)

    pre = _nt(x, dW_ref[...]) + db_ref[...]
    preT = _nt(dW_ref[...], x) + dbT_ref[...]
    cum = _mm(bdf, _nsoftplus(pre))
    cumT = _nt(_nsoftplus(preT), bdf)

    def proj(U0, U1, V0, V1, j):
        h0 = _mm(x, U0[0])
        h1 = _mm(x, U1[0])
        return (ww_ref[2 * j] * _mm(h0, V0[0])
                + ww_ref[2 * j + 1] * _mm(h1, V1[0]))

    q = proj(qU0, qU1, qV0, qV1, 0)
    k = proj(kU0, kU1, kV0, kV1, 1)
    v = proj(vU0, vU1, vV0, vV1, 2)
    g = proj(gU0, gU1, gV0, gV1, 3)
    kv = k * v * (1.0 / (1.0 + jnp.exp(-g)))

    @pl.when(sblk == 0)
    def _init():
        st_ref[...] = jnp.zeros_like(st_ref)

    st_prev = st_ref[...]
    rmask = jax.lax.broadcasted_iota(jnp.int32, (m, 1), 0) < bs

    parts = []
    for h in range(n_heads):
        hsl = slice(h * d_head, (h + 1) * d_head)
        ch = cum[:, h:h + 1]
        ct = cumT[h:h + 1, :]
        A = jnp.where(bd, jnp.exp(ch - ct), 0.0)
        stb = jnp.where(rmask, st_prev[0:1, hsl], st_prev[1:2, hsl])
        sh = _mm(A, kv[:, hsl]) + jnp.exp(ch) * stb
        parts.append(sh)
    states = jnp.concatenate(parts, axis=1)
    st_ref[...] = jnp.concatenate(
        [states[b * bs + bs - 1:b * bs + bs, :] for b in range(bsz)], axis=0)

    y = q * states
    ones = jnp.full((hid, 1), 1.0 / hid, dtype=jnp.float32)
    ms = _mm(y * y, ones)
    y = y * jax.lax.rsqrt(ms + 1e-6) * rms_ref[...]
    for b in range(bsz):
        o_ref[b] = y[b * bs:(b + 1) * bs, :]


def _gate_kernel(x_ref, y_ref, ogW_ref, o_ref):
    x = x_ref[0]
    y = y_ref[0]
    o_ref[0] = y * (1.0 / (1.0 + jnp.exp(-_nt(x, ogW_ref[...]))))


def _proj_kernel(z_ref, outW_ref, o_ref):
    o_ref[0] = _nt(z_ref[0], outW_ref[...])


def kernel(x, q_U, q_V, k_U, k_V, v_U, v_V, q_w, k_w, v_w, gate_w,
           decay_W, decay_b, og_W, out_W, rms_scale):
    bsz, seq_len, dm = x.shape
    np_, _, rank = q_U.shape
    n_heads = decay_W.shape[0]
    hid = q_V.shape[2]
    d_head = hid // n_heads
    bs = _BS
    n_blk = seq_len // bs

    def top2(w):
        tv, ti = jax.lax.top_k(w, 2)
        return ti.astype(jnp.int32), jax.nn.softmax(tv)

    qi, qw2 = top2(q_w)
    ki, kw2 = top2(k_w)
    vi, vw2 = top2(v_w)
    gi, gw2 = top2(gate_w)
    topi = jnp.concatenate([qi, ki, vi, gi])
    ww = jnp.concatenate([qw2, kw2, vw2, gw2])

    db = decay_b.reshape(1, n_heads)
    dbT = decay_b.reshape(n_heads, 1)
    rms = rms_scale.reshape(1, hid)

    def u_spec(j):
        return pl.BlockSpec((1, dm, rank), lambda s, ti, j=j: (ti[j], 0, 0))

    def v_spec(j):
        return pl.BlockSpec((1, rank, hid), lambda s, ti, j=j: (ti[j], 0, 0))

    grid_spec = pltpu.PrefetchScalarGridSpec(
        num_scalar_prefetch=1,
        grid=(n_blk,),
        in_specs=[
            pl.BlockSpec((bsz, bs, dm), lambda s, ti: (0, s, 0)),
            u_spec(0), u_spec(1), u_spec(2), u_spec(3),
            u_spec(4), u_spec(5), u_spec(6), u_spec(7),
            v_spec(0), v_spec(1), v_spec(2), v_spec(3),
            v_spec(4), v_spec(5), v_spec(6), v_spec(7),
            pl.BlockSpec(memory_space=pltpu.SMEM),
            pl.BlockSpec((n_heads, dm), lambda s, ti: (0, 0)),
            pl.BlockSpec((1, n_heads), lambda s, ti: (0, 0)),
            pl.BlockSpec((n_heads, 1), lambda s, ti: (0, 0)),
            pl.BlockSpec((1, hid), lambda s, ti: (0, 0)),
        ],
        out_specs=pl.BlockSpec((bsz, bs, hid), lambda s, ti: (0, s, 0)),
        scratch_shapes=[pltpu.VMEM((bsz, hid), jnp.float32)],
    )

    scan_fn = pl.pallas_call(
        functools.partial(_scan_kernel, n_heads=n_heads, d_head=d_head),
        grid_spec=grid_spec,
        out_shape=jax.ShapeDtypeStruct((bsz, seq_len, hid), jnp.float32),
        compiler_params=pltpu.CompilerParams(
            dimension_semantics=("arbitrary",)),
    )
    y = scan_fn(topi, x,
                q_U, q_U, k_U, k_U, v_U, v_U, v_U, v_U,
                q_V, q_V, k_V, k_V, v_V, v_V, v_V, v_V,
                ww, decay_W, db, dbT, rms)

    bt = _BT
    gate_fn = pl.pallas_call(
        _gate_kernel,
        grid=(bsz, seq_len // bt),
        in_specs=[
            pl.BlockSpec((1, bt, dm), lambda b, s: (b, s, 0)),
            pl.BlockSpec((1, bt, hid), lambda b, s: (b, s, 0)),
            pl.BlockSpec((hid, dm), lambda b, s: (0, 0)),
        ],
        out_specs=pl.BlockSpec((1, bt, hid), lambda b, s: (b, s, 0)),
        out_shape=jax.ShapeDtypeStruct((bsz, seq_len, hid), jnp.float32),
        compiler_params=pltpu.CompilerParams(
            dimension_semantics=("arbitrary", "arbitrary")),
    )
    z = gate_fn(x, y, og_W)

    proj_fn = pl.pallas_call(
        _proj_kernel,
        grid=(bsz, seq_len // bt),
        in_specs=[
            pl.BlockSpec((1, bt, hid), lambda b, s: (b, s, 0)),
            pl.BlockSpec((dm, hid), lambda b, s: (0, 0)),
        ],
        out_specs=pl.BlockSpec((1, bt, dm), lambda b, s: (b, s, 0)),
        out_shape=jax.ShapeDtypeStruct((bsz, seq_len, dm), jnp.float32),
        compiler_params=pltpu.CompilerParams(
            dimension_semantics=("arbitrary", "arbitrary")),
    )
    return proj_fn(z, out_W)


# speedup vs baseline: 1.0381x; 1.0381x over previous
import functools

import jax
import jax.numpy as jnp
from jax.experimental import pallas as pl
from jax.experimental.pallas import tpu as pltpu

_BS = 128
_BT = 256


def _nsoftplus(p):
    return -(jnp.maximum(p, 0.0) + jnp.log1p(jnp.exp(-jnp.abs(p))))


def _nt(a, b):
    return jax.lax.dot_general(a, b, (((1,), (1,)), ((), ())),
                               preferred_element_type=jnp.float32)


def _mm(a, b):
    return jax.lax.dot_general(a, b, (((1,), (0,)), ((), ())),
                               preferred_element_type=jnp.float32)


def _scan_kernel(topi_ref, x_ref,
                 qU0, qU1, kU0, kU1, vU0, vU1, gU0, gU1,
                 qV0, qV1, kV0, kV1, vV0, vV1, gV0, gV1,
                 ww_ref, dW_ref, db_ref, dbT_ref, rms_ref,
                 o_ref, st_ref, *, n_heads, d_head):
    del topi_ref
    sblk = pl.program_id(0)
    bsz = x_ref.shape[0]
    bs = x_ref.shape[1]
    m = bsz * bs
    hid = n_heads * d_head

    x = x_ref[...].reshape(m, x_ref.shape[2])

    rows = jax.lax.broadcasted_iota(jnp.int32, (m, m), 0)
    cols = jax.lax.broadcasted_iota(jnp.int32, (m, m), 1)
    bd = (rows >= cols) & (rows - cols < bs) & ((rows >= bs) == (cols >= bs))
    bdf = bd.astype(jnp.float32)

    pre = _nt(x, dW_ref[...]) + db_ref[...]
    preT = _nt(dW_ref[...], x) + dbT_ref[...]
    cum = _mm(bdf, _nsoftplus(pre))
    cumT = _nt(_nsoftplus(preT), bdf)

    def proj(U0, U1, V0, V1, j):
        h0 = _mm(x, U0[0])
        h1 = _mm(x, U1[0])
        return (ww_ref[2 * j] * _mm(h0, V0[0])
                + ww_ref[2 * j + 1] * _mm(h1, V1[0]))

    q = proj(qU0, qU1, qV0, qV1, 0)
    k = proj(kU0, kU1, kV0, kV1, 1)
    v = proj(vU0, vU1, vV0, vV1, 2)
    g = proj(gU0, gU1, gV0, gV1, 3)
    kv = k * v * (1.0 / (1.0 + jnp.exp(-g)))

    @pl.when(sblk == 0)
    def _init():
        st_ref[...] = jnp.zeros_like(st_ref)

    st_prev = st_ref[...]
    rmask = jax.lax.broadcasted_iota(jnp.int32, (m, 1), 0) < bs

    parts = []
    for h in range(n_heads):
        hsl = slice(h * d_head, (h + 1) * d_head)
        ch = cum[:, h:h + 1]
        ct = cumT[h:h + 1, :]
        A = jnp.where(bd, jnp.exp(ch - ct), 0.0)
        stb = jnp.where(rmask, st_prev[0:1, hsl], st_prev[1:2, hsl])
        sh = _mm(A, kv[:, hsl]) + jnp.exp(ch) * stb
        parts.append(sh)
    states = jnp.concatenate(parts, axis=1)
    st_ref[...] = jnp.concatenate(
        [states[b * bs + bs - 1:b * bs + bs, :] for b in range(bsz)], axis=0)

    y = q * states
    ones = jnp.full((hid, 1), 1.0 / hid, dtype=jnp.float32)
    ms = _mm(y * y, ones)
    y = y * jax.lax.rsqrt(ms + 1e-6) * rms_ref[...]
    for b in range(bsz):
        o_ref[b] = y[b * bs:(b + 1) * bs, :]


def _out_kernel(x_ref, y_ref, ogW_ref, outW_ref, o_ref):
    x = x_ref[0]
    y = y_ref[0]
    og = 1.0 / (1.0 + jnp.exp(-_nt(x, ogW_ref[...])))
    o_ref[0] = _nt(y * og, outW_ref[...])


def kernel(x, q_U, q_V, k_U, k_V, v_U, v_V, q_w, k_w, v_w, gate_w,
           decay_W, decay_b, og_W, out_W, rms_scale):
    bsz, seq_len, dm = x.shape
    np_, _, rank = q_U.shape
    n_heads = decay_W.shape[0]
    hid = q_V.shape[2]
    d_head = hid // n_heads
    bs = _BS
    n_blk = seq_len // bs

    def top2(w):
        tv, ti = jax.lax.top_k(w, 2)
        return ti.astype(jnp.int32), jax.nn.softmax(tv)

    qi, qw2 = top2(q_w)
    ki, kw2 = top2(k_w)
    vi, vw2 = top2(v_w)
    gi, gw2 = top2(gate_w)
    topi = jnp.concatenate([qi, ki, vi, gi])
    ww = jnp.concatenate([qw2, kw2, vw2, gw2])

    db = decay_b.reshape(1, n_heads)
    dbT = decay_b.reshape(n_heads, 1)
    rms = rms_scale.reshape(1, hid)

    def u_spec(j):
        return pl.BlockSpec((1, dm, rank), lambda s, ti, j=j: (ti[j], 0, 0))

    def v_spec(j):
        return pl.BlockSpec((1, rank, hid), lambda s, ti, j=j: (ti[j], 0, 0))

    grid_spec = pltpu.PrefetchScalarGridSpec(
        num_scalar_prefetch=1,
        grid=(n_blk,),
        in_specs=[
            pl.BlockSpec((bsz, bs, dm), lambda s, ti: (0, s, 0)),
            u_spec(0), u_spec(1), u_spec(2), u_spec(3),
            u_spec(4), u_spec(5), u_spec(6), u_spec(7),
            v_spec(0), v_spec(1), v_spec(2), v_spec(3),
            v_spec(4), v_spec(5), v_spec(6), v_spec(7),
            pl.BlockSpec(memory_space=pltpu.SMEM),
            pl.BlockSpec((n_heads, dm), lambda s, ti: (0, 0)),
            pl.BlockSpec((1, n_heads), lambda s, ti: (0, 0)),
            pl.BlockSpec((n_heads, 1), lambda s, ti: (0, 0)),
            pl.BlockSpec((1, hid), lambda s, ti: (0, 0)),
        ],
        out_specs=pl.BlockSpec((bsz, bs, hid), lambda s, ti: (0, s, 0)),
        scratch_shapes=[pltpu.VMEM((bsz, hid), jnp.float32)],
    )

    scan_fn = pl.pallas_call(
        functools.partial(_scan_kernel, n_heads=n_heads, d_head=d_head),
        grid_spec=grid_spec,
        out_shape=jax.ShapeDtypeStruct((bsz, seq_len, hid), jnp.float32),
        compiler_params=pltpu.CompilerParams(
            dimension_semantics=("arbitrary",)),
    )
    y = scan_fn(topi, x,
                q_U, q_U, k_U, k_U, v_U, v_U, v_U, v_U,
                q_V, q_V, k_V, k_V, v_V, v_V, v_V, v_V,
                ww, decay_W, db, dbT, rms)

    bt = _BT
    out_fn = pl.pallas_call(
        _out_kernel,
        grid=(bsz, seq_len // bt),
        in_specs=[
            pl.BlockSpec((1, bt, dm), lambda b, s: (b, s, 0)),
            pl.BlockSpec((1, bt, hid), lambda b, s: (b, s, 0)),
            pl.BlockSpec((hid, dm), lambda b, s: (0, 0)),
            pl.BlockSpec((dm, hid), lambda b, s: (0, 0)),
        ],
        out_specs=pl.BlockSpec((1, bt, dm), lambda b, s: (b, s, 0)),
        out_shape=jax.ShapeDtypeStruct((bsz, seq_len, dm), jnp.float32),
        compiler_params=pltpu.CompilerParams(
            dimension_semantics=("arbitrary", "arbitrary")),
    )
    return out_fn(x, y, og_W, out_W)


# speedup vs baseline: 1.1835x; 1.1400x over previous
import functools

import jax
import jax.numpy as jnp
from jax.experimental import pallas as pl
from jax.experimental.pallas import tpu as pltpu

_BS = 128
_BT = 256


def _nsoftplus(p):
    return -(jnp.maximum(p, 0.0) + jnp.log1p(jnp.exp(-jnp.abs(p))))


def _nt(a, b):
    return jax.lax.dot_general(a, b, (((1,), (1,)), ((), ())),
                               preferred_element_type=jnp.float32)


def _mm(a, b):
    return jax.lax.dot_general(a, b, (((1,), (0,)), ((), ())),
                               preferred_element_type=jnp.float32)


def _pack_kernel(topi_ref,
                 qU0, qU1, kU0, kU1, vU0, vU1, gU0, gU1,
                 qV0, qV1, kV0, kV1, vV0, vV1, gV0, gV1,
                 ww_ref, oU_ref, oV_ref, *, rank):
    del topi_ref
    us = (qU0, qU1, kU0, kU1, vU0, vU1, gU0, gU1)
    vs = (qV0, qV1, kV0, kV1, vV0, vV1, gV0, gV1)
    for j in range(8):
        oU_ref[:, j * rank:(j + 1) * rank] = us[j][0]
        oV_ref[j * rank:(j + 1) * rank, :] = ww_ref[j] * vs[j][0]


def _scan_kernel(x_ref, uall_ref, vall_ref, dW_ref, db_ref, dbT_ref, rms_ref,
                 o_ref, st_ref, *, n_heads, d_head):
    sblk = pl.program_id(0)
    bsz = x_ref.shape[0]
    bs = x_ref.shape[1]
    m = bsz * bs
    hid = n_heads * d_head
    r2 = uall_ref.shape[1] // 4

    x = x_ref[...].reshape(m, x_ref.shape[2])

    rows = jax.lax.broadcasted_iota(jnp.int32, (m, m), 0)
    cols = jax.lax.broadcasted_iota(jnp.int32, (m, m), 1)
    bd = (rows >= cols) & (rows - cols < bs) & ((rows >= bs) == (cols >= bs))
    bdf = bd.astype(jnp.float32)

    pre = _nt(x, dW_ref[...]) + db_ref[...]
    preT = _nt(dW_ref[...], x) + dbT_ref[...]
    cum = _mm(bdf, _nsoftplus(pre))
    cumT = _nt(_nsoftplus(preT), bdf)

    h_all = _mm(x, uall_ref[...])
    q = _mm(h_all[:, 0 * r2:1 * r2], vall_ref[0 * r2:1 * r2, :])
    k = _mm(h_all[:, 1 * r2:2 * r2], vall_ref[1 * r2:2 * r2, :])
    v = _mm(h_all[:, 2 * r2:3 * r2], vall_ref[2 * r2:3 * r2, :])
    g = _mm(h_all[:, 3 * r2:4 * r2], vall_ref[3 * r2:4 * r2, :])
    kv = k * v * (1.0 / (1.0 + jnp.exp(-g)))

    @pl.when(sblk == 0)
    def _init():
        st_ref[...] = jnp.zeros_like(st_ref)

    st_prev = st_ref[...]
    rmask = jax.lax.broadcasted_iota(jnp.int32, (m, 1), 0) < bs

    parts = []
    for h in range(n_heads):
        hsl = slice(h * d_head, (h + 1) * d_head)
        ch = cum[:, h:h + 1]
        ct = cumT[h:h + 1, :]
        A = jnp.where(bd, jnp.exp(ch - ct), 0.0)
        stb = jnp.where(rmask, st_prev[0:1, hsl], st_prev[1:2, hsl])
        sh = _mm(A, kv[:, hsl]) + jnp.exp(ch) * stb
        parts.append(sh)
    states = jnp.concatenate(parts, axis=1)
    st_ref[...] = jnp.concatenate(
        [states[b * bs + bs - 1:b * bs + bs, :] for b in range(bsz)], axis=0)

    y = q * states
    ones = jnp.full((hid, 1), 1.0 / hid, dtype=jnp.float32)
    ms = _mm(y * y, ones)
    y = y * jax.lax.rsqrt(ms + 1e-6) * rms_ref[...]
    for b in range(bsz):
        o_ref[b] = y[b * bs:(b + 1) * bs, :]


def _out_kernel(x_ref, y_ref, ogW_ref, outW_ref, o_ref):
    x = x_ref[0]
    y = y_ref[0]
    og = 1.0 / (1.0 + jnp.exp(-_nt(x, ogW_ref[...])))
    o_ref[0] = _nt(y * og, outW_ref[...])


def kernel(x, q_U, q_V, k_U, k_V, v_U, v_V, q_w, k_w, v_w, gate_w,
           decay_W, decay_b, og_W, out_W, rms_scale):
    bsz, seq_len, dm = x.shape
    np_, _, rank = q_U.shape
    n_heads = decay_W.shape[0]
    hid = q_V.shape[2]
    d_head = hid // n_heads
    bs = _BS
    n_blk = seq_len // bs

    def top2(w):
        tv, ti = jax.lax.top_k(w, 2)
        return ti.astype(jnp.int32), jax.nn.softmax(tv)

    qi, qw2 = top2(q_w)
    ki, kw2 = top2(k_w)
    vi, vw2 = top2(v_w)
    gi, gw2 = top2(gate_w)
    topi = jnp.concatenate([qi, ki, vi, gi])
    ww = jnp.concatenate([qw2, kw2, vw2, gw2])

    db = decay_b.reshape(1, n_heads)
    dbT = decay_b.reshape(n_heads, 1)
    rms = rms_scale.reshape(1, hid)

    def u_spec(j):
        return pl.BlockSpec((1, dm, rank), lambda i, ti, j=j: (ti[j], 0, 0))

    def v_spec(j):
        return pl.BlockSpec((1, rank, hid), lambda i, ti, j=j: (ti[j], 0, 0))

    pack_spec = pltpu.PrefetchScalarGridSpec(
        num_scalar_prefetch=1,
        grid=(1,),
        in_specs=[
            u_spec(0), u_spec(1), u_spec(2), u_spec(3),
            u_spec(4), u_spec(5), u_spec(6), u_spec(7),
            v_spec(0), v_spec(1), v_spec(2), v_spec(3),
            v_spec(4), v_spec(5), v_spec(6), v_spec(7),
            pl.BlockSpec(memory_space=pltpu.SMEM),
        ],
        out_specs=[
            pl.BlockSpec((dm, 8 * rank), lambda i, ti: (0, 0)),
            pl.BlockSpec((8 * rank, hid), lambda i, ti: (0, 0)),
        ],
    )
    pack_fn = pl.pallas_call(
        functools.partial(_pack_kernel, rank=rank),
        grid_spec=pack_spec,
        out_shape=[jax.ShapeDtypeStruct((dm, 8 * rank), jnp.float32),
                   jax.ShapeDtypeStruct((8 * rank, hid), jnp.float32)],
    )
    u_all, v_all = pack_fn(topi,
                           q_U, q_U, k_U, k_U, v_U, v_U, v_U, v_U,
                           q_V, q_V, k_V, k_V, v_V, v_V, v_V, v_V, ww)

    scan_fn = pl.pallas_call(
        functools.partial(_scan_kernel, n_heads=n_heads, d_head=d_head),
        grid=(n_blk,),
        in_specs=[
            pl.BlockSpec((bsz, bs, dm), lambda s: (0, s, 0)),
            pl.BlockSpec((dm, 8 * rank), lambda s: (0, 0)),
            pl.BlockSpec((8 * rank, hid), lambda s: (0, 0)),
            pl.BlockSpec((n_heads, dm), lambda s: (0, 0)),
            pl.BlockSpec((1, n_heads), lambda s: (0, 0)),
            pl.BlockSpec((n_heads, 1), lambda s: (0, 0)),
            pl.BlockSpec((1, hid), lambda s: (0, 0)),
        ],
        out_specs=pl.BlockSpec((bsz, bs, hid), lambda s: (0, s, 0)),
        out_shape=jax.ShapeDtypeStruct((bsz, seq_len, hid), jnp.float32),
        scratch_shapes=[pltpu.VMEM((bsz, hid), jnp.float32)],
        compiler_params=pltpu.CompilerParams(
            dimension_semantics=("arbitrary",)),
    )
    y = scan_fn(x, u_all, v_all, decay_W, db, dbT, rms)

    bt = _BT
    out_fn = pl.pallas_call(
        _out_kernel,
        grid=(bsz, seq_len // bt),
        in_specs=[
            pl.BlockSpec((1, bt, dm), lambda b, s: (b, s, 0)),
            pl.BlockSpec((1, bt, hid), lambda b, s: (b, s, 0)),
            pl.BlockSpec((hid, dm), lambda b, s: (0, 0)),
            pl.BlockSpec((dm, hid), lambda b, s: (0, 0)),
        ],
        out_specs=pl.BlockSpec((1, bt, dm), lambda b, s: (b, s, 0)),
        out_shape=jax.ShapeDtypeStruct((bsz, seq_len, dm), jnp.float32),
        compiler_params=pltpu.CompilerParams(
            dimension_semantics=("arbitrary", "arbitrary")),
    )
    return out_fn(x, y, og_W, out_W)


# speedup vs baseline: 1.2114x; 1.0236x over previous
import functools

import jax
import jax.numpy as jnp
from jax.experimental import pallas as pl
from jax.experimental.pallas import tpu as pltpu

_BS = 128
_BT = 256


def _nsoftplus(p):
    return -(jnp.maximum(p, 0.0) + jnp.log1p(jnp.exp(-jnp.abs(p))))


def _nt(a, b):
    return jax.lax.dot_general(a, b, (((1,), (1,)), ((), ())),
                               preferred_element_type=jnp.float32)


def _mm(a, b):
    return jax.lax.dot_general(a, b, (((1,), (0,)), ((), ())),
                               preferred_element_type=jnp.float32)


def _pack_kernel(topi_ref,
                 qU0, qU1, kU0, kU1, vU0, vU1, gU0, gU1,
                 qV0, qV1, kV0, kV1, vV0, vV1, gV0, gV1,
                 ww_ref, oU_ref, oV_ref, *, rank):
    del topi_ref
    us = (qU0, qU1, kU0, kU1, vU0, vU1, gU0, gU1)
    vs = (qV0, qV1, kV0, kV1, vV0, vV1, gV0, gV1)
    for j in range(8):
        oU_ref[:, j * rank:(j + 1) * rank] = us[j][0]
        oV_ref[j * rank:(j + 1) * rank, :] = ww_ref[j] * vs[j][0]


def _scan_kernel(x_ref, uall_ref, vall_ref, dW_ref, db_ref, rms_ref,
                 o_ref, st_ref, *, n_heads, d_head):
    sblk = pl.program_id(0)
    bsz = x_ref.shape[0]
    bs = x_ref.shape[1]
    m = bsz * bs
    hid = n_heads * d_head
    r2 = uall_ref.shape[1] // 4

    x = x_ref[...].reshape(m, x_ref.shape[2])

    rows = jax.lax.broadcasted_iota(jnp.int32, (m, m), 0)
    cols = jax.lax.broadcasted_iota(jnp.int32, (m, m), 1)
    bd = (rows >= cols) & (rows - cols < bs) & ((rows >= bs) == (cols >= bs))
    bdf = bd.astype(jnp.float32)

    pre = _nt(x, dW_ref[...]) + db_ref[...]
    cum = _mm(bdf, _nsoftplus(pre))
    cumT = jnp.transpose(cum)

    h_all = _mm(x, uall_ref[...])
    q = _mm(h_all[:, 0 * r2:1 * r2], vall_ref[0 * r2:1 * r2, :])
    k = _mm(h_all[:, 1 * r2:2 * r2], vall_ref[1 * r2:2 * r2, :])
    v = _mm(h_all[:, 2 * r2:3 * r2], vall_ref[2 * r2:3 * r2, :])
    g = _mm(h_all[:, 3 * r2:4 * r2], vall_ref[3 * r2:4 * r2, :])
    kv = k * v * (1.0 / (1.0 + jnp.exp(-g)))

    @pl.when(sblk == 0)
    def _init():
        st_ref[...] = jnp.zeros_like(st_ref)

    st_prev = st_ref[...]
    rmask = jax.lax.broadcasted_iota(jnp.int32, (m, 1), 0) < bs

    parts = []
    for h in range(n_heads):
        hsl = slice(h * d_head, (h + 1) * d_head)
        ch = cum[:, h:h + 1]
        ct = cumT[h:h + 1, :]
        A = jnp.where(bd, jnp.exp(ch - ct), 0.0)
        stb = jnp.where(rmask, st_prev[0:1, hsl], st_prev[1:2, hsl])
        sh = _mm(A, kv[:, hsl]) + jnp.exp(ch) * stb
        parts.append(sh)
    states = jnp.concatenate(parts, axis=1)
    st_ref[...] = jnp.concatenate(
        [states[b * bs + bs - 1:b * bs + bs, :] for b in range(bsz)], axis=0)

    y = q * states
    ones = jnp.full((hid, 1), 1.0 / hid, dtype=jnp.float32)
    ms = _mm(y * y, ones)
    y = y * jax.lax.rsqrt(ms + 1e-6) * rms_ref[...]
    for b in range(bsz):
        o_ref[b] = y[b * bs:(b + 1) * bs, :]


def _out_kernel(x_ref, y_ref, ogW_ref, outW_ref, o_ref):
    x = x_ref[0]
    y = y_ref[0]
    og = 1.0 / (1.0 + jnp.exp(-_nt(x, ogW_ref[...])))
    o_ref[0] = _nt(y * og, outW_ref[...])


def kernel(x, q_U, q_V, k_U, k_V, v_U, v_V, q_w, k_w, v_w, gate_w,
           decay_W, decay_b, og_W, out_W, rms_scale):
    bsz, seq_len, dm = x.shape
    np_, _, rank = q_U.shape
    n_heads = decay_W.shape[0]
    hid = q_V.shape[2]
    d_head = hid // n_heads
    bs = _BS
    n_blk = seq_len // bs

    def top2(w):
        tv, ti = jax.lax.top_k(w, 2)
        return ti.astype(jnp.int32), jax.nn.softmax(tv)

    qi, qw2 = top2(q_w)
    ki, kw2 = top2(k_w)
    vi, vw2 = top2(v_w)
    gi, gw2 = top2(gate_w)
    topi = jnp.concatenate([qi, ki, vi, gi])
    ww = jnp.concatenate([qw2, kw2, vw2, gw2])

    db = decay_b.reshape(1, n_heads)
    rms = rms_scale.reshape(1, hid)

    def u_spec(j):
        return pl.BlockSpec((1, dm, rank), lambda i, ti, j=j: (ti[j], 0, 0))

    def v_spec(j):
        return pl.BlockSpec((1, rank, hid), lambda i, ti, j=j: (ti[j], 0, 0))

    pack_spec = pltpu.PrefetchScalarGridSpec(
        num_scalar_prefetch=1,
        grid=(1,),
        in_specs=[
            u_spec(0), u_spec(1), u_spec(2), u_spec(3),
            u_spec(4), u_spec(5), u_spec(6), u_spec(7),
            v_spec(0), v_spec(1), v_spec(2), v_spec(3),
            v_spec(4), v_spec(5), v_spec(6), v_spec(7),
            pl.BlockSpec(memory_space=pltpu.SMEM),
        ],
        out_specs=[
            pl.BlockSpec((dm, 8 * rank), lambda i, ti: (0, 0)),
            pl.BlockSpec((8 * rank, hid), lambda i, ti: (0, 0)),
        ],
    )
    pack_fn = pl.pallas_call(
        functools.partial(_pack_kernel, rank=rank),
        grid_spec=pack_spec,
        out_shape=[jax.ShapeDtypeStruct((dm, 8 * rank), jnp.float32),
                   jax.ShapeDtypeStruct((8 * rank, hid), jnp.float32)],
    )
    u_all, v_all = pack_fn(topi,
                           q_U, q_U, k_U, k_U, v_U, v_U, v_U, v_U,
                           q_V, q_V, k_V, k_V, v_V, v_V, v_V, v_V, ww)

    scan_fn = pl.pallas_call(
        functools.partial(_scan_kernel, n_heads=n_heads, d_head=d_head),
        grid=(n_blk,),
        in_specs=[
            pl.BlockSpec((bsz, bs, dm), lambda s: (0, s, 0)),
            pl.BlockSpec((dm, 8 * rank), lambda s: (0, 0)),
            pl.BlockSpec((8 * rank, hid), lambda s: (0, 0)),
            pl.BlockSpec((n_heads, dm), lambda s: (0, 0)),
            pl.BlockSpec((1, n_heads), lambda s: (0, 0)),
            pl.BlockSpec((1, hid), lambda s: (0, 0)),
        ],
        out_specs=pl.BlockSpec((bsz, bs, hid), lambda s: (0, s, 0)),
        out_shape=jax.ShapeDtypeStruct((bsz, seq_len, hid), jnp.float32),
        scratch_shapes=[pltpu.VMEM((bsz, hid), jnp.float32)],
        compiler_params=pltpu.CompilerParams(
            dimension_semantics=("arbitrary",)),
    )
    y = scan_fn(x, u_all, v_all, decay_W, db, rms)

    bt = _BT
    out_fn = pl.pallas_call(
        _out_kernel,
        grid=(bsz, seq_len // bt),
        in_specs=[
            pl.BlockSpec((1, bt, dm), lambda b, s: (b, s, 0)),
            pl.BlockSpec((1, bt, hid), lambda b, s: (b, s, 0)),
            pl.BlockSpec((hid, dm), lambda b, s: (0, 0)),
            pl.BlockSpec((dm, hid), lambda b, s: (0, 0)),
        ],
        out_specs=pl.BlockSpec((1, bt, dm), lambda b, s: (b, s, 0)),
        out_shape=jax.ShapeDtypeStruct((bsz, seq_len, dm), jnp.float32),
        compiler_params=pltpu.CompilerParams(
            dimension_semantics=("arbitrary", "arbitrary")),
    )
    return out_fn(x, y, og_W, out_W)


# speedup vs baseline: 1.2166x; 1.0043x over previous
import functools

import jax
import jax.numpy as jnp
from jax.experimental import pallas as pl
from jax.experimental.pallas import tpu as pltpu

_BS = 128
_BT = 256


def _nsoftplus(p):
    return -(jnp.maximum(p, 0.0) + jnp.log1p(jnp.exp(-jnp.abs(p))))


def _nt(a, b):
    return jax.lax.dot_general(a, b, (((1,), (1,)), ((), ())),
                               preferred_element_type=jnp.float32)


def _mm(a, b):
    return jax.lax.dot_general(a, b, (((1,), (0,)), ((), ())),
                               preferred_element_type=jnp.float32)


def _pack_kernel(topi_ref,
                 qU0, qU1, kU0, kU1, vU0, vU1, gU0, gU1,
                 qV0, qV1, kV0, kV1, vV0, vV1, gV0, gV1,
                 ww_ref, oU_ref, oV_ref, *, rank):
    del topi_ref
    del ww_ref
    us = (qU0, qU1, kU0, kU1, vU0, vU1, gU0, gU1)
    vs = (qV0, qV1, kV0, kV1, vV0, vV1, gV0, gV1)
    for j in range(8):
        oU_ref[:, j * rank:(j + 1) * rank] = us[j][0]
        oV_ref[j * rank:(j + 1) * rank, :] = vs[j][0]


def _scan_kernel(x_ref, uall_ref, vall_ref, wrep_ref, dW_ref, db_ref, rms_ref,
                 o_ref, st_ref, *, n_heads, d_head):
    sblk = pl.program_id(0)
    bsz = x_ref.shape[0]
    bs = x_ref.shape[1]
    m = bsz * bs
    hid = n_heads * d_head
    r2 = uall_ref.shape[1] // 4

    x = x_ref[...].reshape(m, x_ref.shape[2])

    rows = jax.lax.broadcasted_iota(jnp.int32, (m, m), 0)
    cols = jax.lax.broadcasted_iota(jnp.int32, (m, m), 1)
    bd = (rows >= cols) & (rows - cols < bs) & ((rows >= bs) == (cols >= bs))
    bdf = bd.astype(jnp.float32)

    pre = _nt(x, dW_ref[...]) + db_ref[...]
    cum = _mm(bdf, _nsoftplus(pre))
    cumT = jnp.transpose(cum)

    h_all = _mm(x, uall_ref[...]) * wrep_ref[...]
    q = _mm(h_all[:, 0 * r2:1 * r2], vall_ref[0 * r2:1 * r2, :])
    k = _mm(h_all[:, 1 * r2:2 * r2], vall_ref[1 * r2:2 * r2, :])
    v = _mm(h_all[:, 2 * r2:3 * r2], vall_ref[2 * r2:3 * r2, :])
    g = _mm(h_all[:, 3 * r2:4 * r2], vall_ref[3 * r2:4 * r2, :])
    kv = k * v * (1.0 / (1.0 + jnp.exp(-g)))

    @pl.when(sblk == 0)
    def _init():
        st_ref[...] = jnp.zeros_like(st_ref)

    st_prev = st_ref[...]
    rmask = jax.lax.broadcasted_iota(jnp.int32, (m, 1), 0) < bs

    parts = []
    for h in range(n_heads):
        hsl = slice(h * d_head, (h + 1) * d_head)
        ch = cum[:, h:h + 1]
        ct = cumT[h:h + 1, :]
        A = jnp.where(bd, jnp.exp(ch - ct), 0.0)
        stb = jnp.where(rmask, st_prev[0:1, hsl], st_prev[1:2, hsl])
        sh = _mm(A, kv[:, hsl]) + jnp.exp(ch) * stb
        parts.append(sh)
    states = jnp.concatenate(parts, axis=1)
    st_ref[...] = jnp.concatenate(
        [states[b * bs + bs - 1:b * bs + bs, :] for b in range(bsz)], axis=0)

    y = q * states
    ones = jnp.full((hid, 1), 1.0 / hid, dtype=jnp.float32)
    ms = _mm(y * y, ones)
    y = y * jax.lax.rsqrt(ms + 1e-6) * rms_ref[...]
    for b in range(bsz):
        o_ref[b] = y[b * bs:(b + 1) * bs, :]


def _out_kernel(x_ref, y_ref, ogW_ref, outW_ref, o_ref):
    x = x_ref[0]
    y = y_ref[0]
    og = 1.0 / (1.0 + jnp.exp(-_nt(x, ogW_ref[...])))
    o_ref[0] = _nt(y * og, outW_ref[...])


def kernel(x, q_U, q_V, k_U, k_V, v_U, v_V, q_w, k_w, v_w, gate_w,
           decay_W, decay_b, og_W, out_W, rms_scale):
    bsz, seq_len, dm = x.shape
    np_, _, rank = q_U.shape
    n_heads = decay_W.shape[0]
    hid = q_V.shape[2]
    d_head = hid // n_heads
    bs = _BS
    n_blk = seq_len // bs

    def top2(w):
        tv, ti = jax.lax.top_k(w, 2)
        return ti.astype(jnp.int32), jax.nn.softmax(tv)

    qi, qw2 = top2(q_w)
    ki, kw2 = top2(k_w)
    vi, vw2 = top2(v_w)
    gi, gw2 = top2(gate_w)
    topi = jnp.concatenate([qi, ki, vi, gi])
    ww = jnp.concatenate([qw2, kw2, vw2, gw2])

    db = decay_b.reshape(1, n_heads)
    rms = rms_scale.reshape(1, hid)
    wrep = jnp.repeat(ww, rank).reshape(1, 8 * rank)

    def u_spec(j):
        return pl.BlockSpec((1, dm, rank), lambda i, ti, j=j: (ti[j], 0, 0))

    def v_spec(j):
        return pl.BlockSpec((1, rank, hid), lambda i, ti, j=j: (ti[j], 0, 0))

    pack_spec = pltpu.PrefetchScalarGridSpec(
        num_scalar_prefetch=1,
        grid=(1,),
        in_specs=[
            u_spec(0), u_spec(1), u_spec(2), u_spec(3),
            u_spec(4), u_spec(5), u_spec(6), u_spec(7),
            v_spec(0), v_spec(1), v_spec(2), v_spec(3),
            v_spec(4), v_spec(5), v_spec(6), v_spec(7),
            pl.BlockSpec(memory_space=pltpu.SMEM),
        ],
        out_specs=[
            pl.BlockSpec((dm, 8 * rank), lambda i, ti: (0, 0)),
            pl.BlockSpec((8 * rank, hid), lambda i, ti: (0, 0)),
        ],
    )
    pack_fn = pl.pallas_call(
        functools.partial(_pack_kernel, rank=rank),
        grid_spec=pack_spec,
        out_shape=[jax.ShapeDtypeStruct((dm, 8 * rank), jnp.float32),
                   jax.ShapeDtypeStruct((8 * rank, hid), jnp.float32)],
    )
    u_all, v_all = pack_fn(topi,
                           q_U, q_U, k_U, k_U, v_U, v_U, v_U, v_U,
                           q_V, q_V, k_V, k_V, v_V, v_V, v_V, v_V, ww)

    scan_fn = pl.pallas_call(
        functools.partial(_scan_kernel, n_heads=n_heads, d_head=d_head),
        grid=(n_blk,),
        in_specs=[
            pl.BlockSpec((bsz, bs, dm), lambda s: (0, s, 0)),
            pl.BlockSpec((dm, 8 * rank), lambda s: (0, 0)),
            pl.BlockSpec((8 * rank, hid), lambda s: (0, 0)),
            pl.BlockSpec((1, 8 * rank), lambda s: (0, 0)),
            pl.BlockSpec((n_heads, dm), lambda s: (0, 0)),
            pl.BlockSpec((1, n_heads), lambda s: (0, 0)),
            pl.BlockSpec((1, hid), lambda s: (0, 0)),
        ],
        out_specs=pl.BlockSpec((bsz, bs, hid), lambda s: (0, s, 0)),
        out_shape=jax.ShapeDtypeStruct((bsz, seq_len, hid), jnp.float32),
        scratch_shapes=[pltpu.VMEM((bsz, hid), jnp.float32)],
        compiler_params=pltpu.CompilerParams(
            dimension_semantics=("arbitrary",)),
    )
    y = scan_fn(x, u_all, v_all, wrep, decay_W, db, rms)

    bt = _BT
    out_fn = pl.pallas_call(
        _out_kernel,
        grid=(bsz, seq_len // bt),
        in_specs=[
            pl.BlockSpec((1, bt, dm), lambda b, s: (b, s, 0)),
            pl.BlockSpec((1, bt, hid), lambda b, s: (b, s, 0)),
            pl.BlockSpec((hid, dm), lambda b, s: (0, 0)),
            pl.BlockSpec((dm, hid), lambda b, s: (0, 0)),
        ],
        out_specs=pl.BlockSpec((1, bt, dm), lambda b, s: (b, s, 0)),
        out_shape=jax.ShapeDtypeStruct((bsz, seq_len, dm), jnp.float32),
        compiler_params=pltpu.CompilerParams(
            dimension_semantics=("arbitrary", "arbitrary")),
    )
    return out_fn(x, y, og_W, out_W)
